# Initial kernel scaffold; baseline (speedup 1.0000x reference)
import jax
import jax.numpy as jnp
from jax.experimental import pallas as pl


def kernel(x, emb_proto, W_len, b_len, emb_flags, W_iat, b_iat, emb_dir, W_fuse, b_fuse, gamma, beta):
    raise NotImplementedError("write your pallas kernel here")


# speedup vs baseline: 1.3983x; 1.3983x over previous
import functools

import jax
import jax.numpy as jnp
from jax import lax
from jax.experimental import pallas as pl
from jax.experimental.pallas import tpu as pltpu
from jax.experimental.pallas import tpu_sc as plsc

_B, _S, _D = 4096, 50, 256
_N = _B * _S
_NW = 32
_TPW = _N // _NW
_XB = 320
_OB = 8
_F32 = jnp.float32


def _dot(x, y):
    return lax.dot_general(x, y, (((1,), (0,)), ((), ())),
                           precision=lax.Precision.HIGHEST,
                           preferred_element_type=jnp.float32)


def _dott(x, y):
    return lax.dot_general(x, y, (((1,), (1,)), ((), ())),
                           precision=lax.Precision.HIGHEST,
                           preferred_element_type=jnp.float32)


def _prep_body(ep, wl, bl, ef, wi, bi, ed, wf, bf, g,
               tp_o, tfd_o, pf_o, qtab_o, uv_o, wv_o, cst_o):
    wf_ = wf[...]
    Tp = _dot(ep[...], wf_[0:16])
    U = _dot(wl[...], wf_[16:48])
    Cl = _dot(bl[...], wf_[16:48])
    Tf = _dot(ef[...], wf_[48:64])
    Wv = _dot(wi[...], wf_[64:96])
    Ci = _dot(bi[...], wf_[64:96])
    Td = _dot(ed[...], wf_[96:104])
    C = Cl + Ci + bf[...]

    ctr = lambda m: m - jnp.mean(m, axis=1, keepdims=True)
    P = ctr(Tp)
    F = ctr(Tf)
    Dc = ctr(Td)
    Up = ctr(U)
    Wp = ctr(Wv)
    Cp = ctr(C)

    rows = lax.broadcasted_iota(jnp.int32, (128, 64), 0)
    cols = lax.broadcasted_iota(jnp.int32, (128, 64), 1)
    oh1 = jnp.where(cols == rows // 2, 1.0, 0.0).astype(jnp.float32)
    rows2 = lax.broadcasted_iota(jnp.int32, (128, 2), 0)
    cols2 = lax.broadcasted_iota(jnp.int32, (128, 2), 1)
    oh2 = jnp.where(cols2 == rows2 % 2, 1.0, 0.0).astype(jnp.float32)
    Q = _dot(oh1, F) + _dot(oh2, Dc) + Cp

    gv = g[...]
    tp_o[...] = P * gv
    tfd_o[...] = Q * gv
    uv_o[...] = Up * gv
    wv_o[...] = Wp * gv
    rsum = lambda m: jnp.sum(m, axis=1, keepdims=True)
    pp = rsum(P * P)
    pu = 2.0 * rsum(P * Up)
    pw = 2.0 * rsum(P * Wp)
    pd_a = 2.0 * rsum(P * (Dc[0:1] + Cp))
    pd_d = 2.0 * rsum(P * (Dc[1:2] - Dc[0:1]))
    pf_o[...] = jnp.concatenate(
        [2.0 * _dott(P, F), pp, pu, pw, pd_a, pd_d,
         jnp.zeros((256, 11), jnp.float32)], axis=1)
    qq = rsum(Q * Q)
    qu = 2.0 * rsum(Q * Up)
    qw = 2.0 * rsum(Q * Wp)
    qtab_o[...] = jnp.concatenate(
        [qq, qu, qw, jnp.zeros((128, 13), jnp.float32)], axis=1)
    uu = jnp.full((1, 1), jnp.sum(Up * Up), jnp.float32)
    ww = jnp.full((1, 1), jnp.sum(Wp * Wp), jnp.float32)
    uw = jnp.full((1, 1), 2.0 * jnp.sum(Up * Wp), jnp.float32)
    cst_o[...] = jnp.concatenate(
        [uu, ww, uw, jnp.zeros((1, 13), jnp.float32)], axis=1)


_PREP_SHAPES = [(256, 256), (128, 256), (256, 80), (128, 16),
                (1, 256), (1, 256), (1, 16)]


def _precompute(ep, wl, bl, ef, wi, bi, ed, wf, bf, g):
    return pl.pallas_call(
        _prep_body,
        out_shape=[jax.ShapeDtypeStruct(s, jnp.float32) for s in _PREP_SHAPES],
    )(ep, wl, bl, ef, wi, bi, ed, wf, bf, g)


def _sc_body(xflat, tp, tfd, pf, qtab, uv, wv, cst, beta, out,
             tp_v, tfd_v, pf_v, qtab_v, uv_v, wv_v, cst_v, beta_v,
             xbuf, ob0, ob1, sem0, sem1):
    wid = lax.axis_index("s") * 2 + lax.axis_index("c")
    base = wid * _TPW

    for src, dst in ((tp, tp_v), (tfd, tfd_v), (pf, pf_v),
                     (qtab, qtab_v), (uv, uv_v), (wv, wv_v), (cst, cst_v),
                     (beta, beta_v)):
        pltpu.sync_copy(src, dst)
    cv = cst_v[pl.ds(0, 16)]
    uu_s = cv[0]
    ww_s = cv[1]
    uw_s = cv[2]

    def token(tok_local, obuf, trow):
        xv = xbuf[pl.ds(tok_local * 5, 16)]
        x1 = xv[1]
        x3 = xv[3]

        def _floor_i32(v):
            r = v.astype(jnp.int32)
            return r - (r.astype(jnp.float32) > v).astype(jnp.int32)

        p = _floor_i32(xv[0])
        f = _floor_i32(xv[2])
        d = _floor_i32(xv[4])
        e = f * 2 + d
        pb = p * 80
        ptr = pf_v[pl.ds(pb + 64, 16)]
        qtr = qtab_v[pl.ds(e * 16, 16)]
        pf_s = pf_v[pl.ds(pb + f, 16)][0]
        s = (ptr[0] + qtr[0] + pf_s + ptr[3] + d.astype(jnp.float32) * ptr[4]
             + x1 * (ptr[1] + qtr[1]) + x3 * (ptr[2] + qtr[2])
             + x1 * x1 * uu_s + x3 * x3 * ww_s + x1 * x3 * uw_s)
        var = s * (1.0 / 256.0) + 1e-5
        iv = jnp.int32(0x5F3759DF) - lax.shift_right_arithmetic(
            lax.bitcast_convert_type(var, jnp.int32), 1)
        y = lax.bitcast_convert_type(iv, jnp.float32)
        y = y * (1.5 - 0.5 * var * y * y)
        y = y * (1.5 - 0.5 * var * y * y)
        y = y * (1.5 - 0.5 * var * y * y)
        ai = x1 * y
        bi = x3 * y
        tpb = p * 256
        eb = e * 256
        for j in range(16):
            sl = pl.ds(j * 16, 16)
            r = (y * (tp_v[pl.ds(tpb + j * 16, 16)]
                      + tfd_v[pl.ds(eb + j * 16, 16)])
                 + ai * uv_v[sl] + bi * wv_v[sl] + beta_v[sl])
            obuf[trow, sl] = r

    nxc = _TPW // _XB
    nobp = _XB // _OB // 2

    def xchunk(cx, carry):
        pltpu.sync_copy(
            xflat.at[pl.ds((base + cx * _XB) * 5, _XB * 5)],
            xbuf.at[pl.ds(0, _XB * 5)])

        def obpair(i, carry2):
            for ph, (obuf, sem) in enumerate(((ob0, sem0), (ob1, sem1))):
                obi = i * 2 + ph
                rowb = base + cx * _XB + obi * _OB
                gb = cx * (_XB // _OB) + obi

                @pl.when(gb >= 2)
                def _wait():
                    pltpu.make_async_copy(
                        obuf, out.at[pl.ds(rowb, _OB)], sem).wait()

                def tl(t, carry3):
                    token(obi * _OB + t, obuf, t)
                    return carry3

                lax.fori_loop(0, _OB, tl, 0)
                pltpu.async_copy(obuf, out.at[pl.ds(rowb, _OB)], sem)
            return carry2

        lax.fori_loop(0, nobp, obpair, 0)
        return carry

    lax.fori_loop(0, nxc, xchunk, 0)
    pltpu.make_async_copy(ob0, out.at[pl.ds(base, _OB)], sem0).wait()
    pltpu.make_async_copy(ob1, out.at[pl.ds(base, _OB)], sem1).wait()


_SC_SCRATCH = [
    pltpu.VMEM((256 * 256,), _F32), pltpu.VMEM((128 * 256,), _F32),
    pltpu.VMEM((256 * 80,), _F32), pltpu.VMEM((128 * 16,), _F32),
    pltpu.VMEM((256,), _F32), pltpu.VMEM((256,), _F32),
    pltpu.VMEM((16,), _F32), pltpu.VMEM((256,), _F32),
    pltpu.VMEM((_XB * 5 + 16,), _F32),
    pltpu.VMEM((_OB, _D), _F32), pltpu.VMEM((_OB, _D), _F32),
    pltpu.SemaphoreType.DMA, pltpu.SemaphoreType.DMA,
]


@functools.cache
def _get_sc_embed():
    return functools.partial(
        pl.kernel,
        out_type=jax.ShapeDtypeStruct((_N, _D), jnp.float32),
        mesh=plsc.VectorSubcoreMesh(core_axis_name="c", subcore_axis_name="s"),
        scratch_types=_SC_SCRATCH,
    )(_sc_body)


def kernel(x, emb_proto, W_len, b_len, emb_flags, W_iat, b_iat, emb_dir,
           W_fuse, b_fuse, gamma, beta):
    pre = _precompute(emb_proto, W_len, b_len.reshape(1, 32), emb_flags,
                      W_iat, b_iat.reshape(1, 32), emb_dir, W_fuse,
                      b_fuse.reshape(1, 256), gamma.reshape(1, 256))
    flat = [t.reshape(-1) for t in pre]
    out = _get_sc_embed()(x.reshape(-1), *flat, beta)
    return out.reshape(_B, _S, _D)
